# Initial kernel scaffold; baseline (speedup 1.0000x reference)
import functools

import jax
import jax.numpy as jnp
from jax import lax
from jax.experimental import pallas as pl
from jax.experimental.pallas import tpu as pltpu
from jax.experimental.pallas import tpu_sc as plsc

B = 16
P = 1024
K = 20
KP = 24
NP = B * P
E = NP * K
EPS = 1e-5
F32 = jnp.float32

TPK = 256
TPE = 2048
NBP = NP // TPE
TPP = 256

_NC, _NS = 2, 16
_NW = _NC * _NS


def _popmin(d2, iota, nkeep):
    cols = []
    for _ in range(nkeep):
        m = jnp.min(d2, axis=1, keepdims=True)
        cand = jnp.where(d2 == m, iota, d2.shape[1])
        j = jnp.min(cand, axis=1, keepdims=True)
        cols.append(j)
        d2 = jnp.where(iota == j, jnp.inf, d2)
    return cols


def _knn1_body(posp_ref, post_ref, idx_ref):
    b = pl.program_id(0)
    x = posp_ref[...]
    xt = post_ref[0]
    sq_r = jnp.sum(x * x, axis=1, keepdims=True)
    sq_c = jnp.sum(xt * xt, axis=0, keepdims=True)
    d2 = sq_r + sq_c - 2.0 * jnp.dot(x, xt, preferred_element_type=F32)
    iota = lax.broadcasted_iota(jnp.int32, (TPK, P), 1)
    cols = _popmin(d2, iota, K)
    idx_ref[...] = jnp.concatenate(cols, axis=1) + b * P


def _knn2_body(x_ref, xall_ref, idx_ref):
    b = pl.program_id(0)
    p = pl.program_id(1)
    x = x_ref[0]
    xa = xall_ref[0]
    dn = (((1,), (1,)), ((), ()))
    sq_r = jnp.sum(x * x, axis=1, keepdims=True)
    ones = jnp.ones((8, 64), F32)
    sq_c = lax.dot_general(ones, xa * xa, dn, preferred_element_type=F32)[0:1]
    d2 = sq_r + sq_c - 2.0 * lax.dot_general(x, xa, dn, preferred_element_type=F32)
    iota = lax.broadcasted_iota(jnp.int32, (TPK, P), 1)
    cols = _popmin(d2, iota, K)
    knn = jnp.concatenate(cols, axis=1) + b * P
    self_i = (b * P + p * TPK
              + lax.broadcasted_iota(jnp.int32, (TPK, 1), 0))
    pad = jnp.concatenate([self_i] * (KP - K), axis=1)
    idx_ref[...] = jnp.concatenate([knn, pad], axis=1)


def _sc_gather_rows(table, idx):
    ew = E // _NW
    ch = 128
    nch = ew // ch
    mesh = plsc.VectorSubcoreMesh(core_axis_name="c", subcore_axis_name="s")

    @functools.partial(
        pl.kernel, mesh=mesh,
        out_type=jax.ShapeDtypeStruct((E, 16), F32),
        scratch_types=[pltpu.VMEM((ch,), jnp.int32),
                       pltpu.VMEM((ch, 16), F32),
                       pltpu.SemaphoreType.DMA],
    )
    def run(table_hbm, idx_hbm, out_hbm, idx_v, rows_v, sem):
        wid = lax.axis_index("s") * _NC + lax.axis_index("c")
        base = wid * ew

        def body(i, carry):
            off = base + i * ch
            pltpu.sync_copy(idx_hbm.at[pl.ds(off, ch)], idx_v)
            pltpu.async_copy(table_hbm.at[idx_v], rows_v, sem).wait()
            pltpu.sync_copy(rows_v, out_hbm.at[pl.ds(off, ch)])
            return carry

        lax.fori_loop(0, nch, body, 0)

    return run(table, idx)


def _sc_gather_max(y, idx):
    pw = NP // _NW
    mesh = plsc.VectorSubcoreMesh(core_axis_name="c", subcore_axis_name="s")

    @functools.partial(
        pl.kernel, mesh=mesh,
        out_type=jax.ShapeDtypeStruct((NP, 128), F32),
        scratch_types=[pltpu.VMEM((KP,), jnp.int32),
                       pltpu.VMEM((KP, 128), F32),
                       pltpu.VMEM((128,), F32),
                       pltpu.SemaphoreType.DMA],
    )
    def run(y_hbm, idx_hbm, out_hbm, idx_v, rows_v, row_v, sem):
        wid = lax.axis_index("s") * _NC + lax.axis_index("c")
        base = wid * pw

        def body(p, carry):
            pt = base + p
            pltpu.sync_copy(idx_hbm.at[pl.ds(pt * KP, KP)], idx_v)
            pltpu.async_copy(y_hbm.at[idx_v], rows_v, sem).wait()
            for c in range(8):
                v = rows_v[0, pl.ds(c * 16, 16)]
                for r in range(1, KP):
                    v = jnp.maximum(v, rows_v[r, pl.ds(c * 16, 16)])
                row_v[pl.ds(c * 16, 16)] = v
            pltpu.sync_copy(row_v, out_hbm.at[pt])
            return carry

        lax.fori_loop(0, pw, body, 0)

    return run(y, idx)


def _stats_update(st_ref, h, g):
    st = jnp.concatenate([jnp.sum(h, axis=0, keepdims=True),
                          jnp.sum(h * h, axis=0, keepdims=True)], axis=0)

    @pl.when(g == 0)
    def _():
        st_ref[...] = st

    @pl.when(g != 0)
    def _():
        st_ref[...] = st_ref[...] + st


def _norm_consts(st):
    mu = st[0:1] * (1.0 / E)
    var = st[1:2] * (1.0 / E) - mu * mu
    return mu, lax.rsqrt(var + EPS)


def _edge1_body(xi_ref, xj_ref, wa_ref, wb_ref, b1_ref, h1_ref, st_ref):
    g = pl.program_id(0)
    h1 = (jnp.dot(xi_ref[...], wa_ref[...], preferred_element_type=F32)
          + jnp.dot(xj_ref[...], wb_ref[...], preferred_element_type=F32)
          + b1_ref[...])
    h1_ref[...] = h1
    _stats_update(st_ref, h1, g)


def _edge2_body(h1_ref, st1_ref, w2_ref, b2_ref, h2_ref, st_ref):
    g = pl.program_id(0)
    mu, rs = _norm_consts(st1_ref[...])
    hn = jnp.maximum((h1_ref[...] - mu) * rs, 0.0)
    h2 = jnp.dot(hn, w2_ref[...], preferred_element_type=F32) + b2_ref[...]
    h2_ref[...] = h2
    _stats_update(st_ref, h2, g)


def _edge3_body(h2_ref, st2_ref, w3_ref, b3_ref, w4b_ref, w4d_ref, b4_ref,
                x1_ref, y_ref, u_ref):
    mu, rs = _norm_consts(st2_ref[...])
    acc = jnp.full((TPP, 64), -jnp.inf, F32)
    for k in range(K):
        hn = jnp.maximum((h2_ref[k] - mu) * rs, 0.0)
        v = jnp.dot(hn, w3_ref[...], preferred_element_type=F32) + b3_ref[...]
        acc = jnp.maximum(acc, v)
    x1_ref[...] = acc
    y_ref[...] = jnp.dot(acc, w4b_ref[...], preferred_element_type=F32)
    u_ref[...] = jnp.dot(acc, w4d_ref[...], preferred_element_type=F32) + b4_ref[...]


def _pool_body(x1_ref, u_ref, m_ref, w5a_ref, w5b_ref, b5_ref, out_ref):
    p = pl.program_id(1)
    t = (jnp.dot(x1_ref[...], w5a_ref[...], preferred_element_type=F32)
         + jnp.dot(u_ref[...] + m_ref[...], w5b_ref[...], preferred_element_type=F32)
         + b5_ref[...])
    v = jnp.max(t, axis=0, keepdims=True)

    @pl.when(p == 0)
    def _():
        out_ref[...] = v

    @pl.when(p != 0)
    def _():
        out_ref[...] = jnp.maximum(out_ref[...], v)


def _bn_relu_rows(h):
    mu = jnp.mean(h, axis=0, keepdims=True)
    var = jnp.mean((h - mu) ** 2, axis=0, keepdims=True)
    return jnp.maximum((h - mu) * lax.rsqrt(var + EPS), 0.0)


def _head_body(z_ref, w6_ref, b6_ref, w7_ref, b7_ref, w8_ref, b8_ref, o_ref):
    h = jnp.dot(z_ref[...], w6_ref[...], preferred_element_type=F32) + b6_ref[...]
    h = _bn_relu_rows(h)
    h = jnp.dot(h, w7_ref[...], preferred_element_type=F32) + b7_ref[...]
    h = _bn_relu_rows(h)
    h = jnp.dot(h, w8_ref[...], preferred_element_type=F32) + b8_ref[...]
    mx = jnp.max(h, axis=1, keepdims=True)
    e = jnp.exp(h - mx)
    o_ref[...] = h - mx - jnp.log(jnp.sum(e, axis=1, keepdims=True))


def kernel(pos, batch, W1, b1, W2, b2, W3, b3, W4, b4, W5, b5, W6, b6, W7, b7, W8, b8):
    del batch
    posp = jnp.pad(pos, ((0, 0), (0, 13)))
    post = jnp.pad(pos.reshape(B, P, 3).transpose(0, 2, 1),
                   ((0, 0), (0, 13), (0, 0)))
    w1a = jnp.pad(W1[0:3], ((0, 13), (0, 0)))
    w1b = jnp.pad(W1[3:6], ((0, 13), (0, 0)))
    wa = w1a - w1b
    w4a, w4b = W4[:64], W4[64:]
    w4d = w4a - w4b
    w5a, w5b = W5[:64], W5[64:]

    nb = P // TPK

    idx1 = pl.pallas_call(
        _knn1_body, grid=(B, nb),
        in_specs=[pl.BlockSpec((TPK, 16), lambda b, p: (b * nb + p, 0)),
                  pl.BlockSpec((1, 16, P), lambda b, p: (b, 0, 0))],
        out_specs=pl.BlockSpec((TPK, K), lambda b, p: (b * nb + p, 0)),
        out_shape=jax.ShapeDtypeStruct((NP, K), jnp.int32),
    )(posp, post)

    idx_sm = idx1.T.reshape(-1)
    xj = _sc_gather_rows(posp, idx_sm)

    h1, st1 = pl.pallas_call(
        _edge1_body, grid=(E // TPE,),
        in_specs=[pl.BlockSpec((TPE, 16), lambda g: (g % NBP, 0)),
                  pl.BlockSpec((TPE, 16), lambda g: (g, 0)),
                  pl.BlockSpec((16, 64), lambda g: (0, 0)),
                  pl.BlockSpec((16, 64), lambda g: (0, 0)),
                  pl.BlockSpec((1, 64), lambda g: (0, 0))],
        out_specs=[pl.BlockSpec((TPE, 64), lambda g: (g, 0)),
                   pl.BlockSpec((2, 64), lambda g: (0, 0))],
        out_shape=[jax.ShapeDtypeStruct((E, 64), F32),
                   jax.ShapeDtypeStruct((2, 64), F32)],
    )(posp, xj, wa, w1b, b1.reshape(1, 64))

    h2, st2 = pl.pallas_call(
        _edge2_body, grid=(E // TPE,),
        in_specs=[pl.BlockSpec((TPE, 64), lambda g: (g, 0)),
                  pl.BlockSpec((2, 64), lambda g: (0, 0)),
                  pl.BlockSpec((64, 64), lambda g: (0, 0)),
                  pl.BlockSpec((1, 64), lambda g: (0, 0))],
        out_specs=[pl.BlockSpec((TPE, 64), lambda g: (g, 0)),
                   pl.BlockSpec((2, 64), lambda g: (0, 0))],
        out_shape=[jax.ShapeDtypeStruct((E, 64), F32),
                   jax.ShapeDtypeStruct((2, 64), F32)],
    )(h1, st1, W2, b2.reshape(1, 64))

    x1, y, u = pl.pallas_call(
        _edge3_body, grid=(NP // TPP,),
        in_specs=[pl.BlockSpec((K, TPP, 64), lambda p: (0, p, 0)),
                  pl.BlockSpec((2, 64), lambda p: (0, 0)),
                  pl.BlockSpec((64, 64), lambda p: (0, 0)),
                  pl.BlockSpec((1, 64), lambda p: (0, 0)),
                  pl.BlockSpec((64, 128), lambda p: (0, 0)),
                  pl.BlockSpec((64, 128), lambda p: (0, 0)),
                  pl.BlockSpec((1, 128), lambda p: (0, 0))],
        out_specs=[pl.BlockSpec((TPP, 64), lambda p: (p, 0)),
                   pl.BlockSpec((TPP, 128), lambda p: (p, 0)),
                   pl.BlockSpec((TPP, 128), lambda p: (p, 0))],
        out_shape=[jax.ShapeDtypeStruct((NP, 64), F32),
                   jax.ShapeDtypeStruct((NP, 128), F32),
                   jax.ShapeDtypeStruct((NP, 128), F32)],
    )(h2.reshape(K, NP, 64), st2, W3, b3.reshape(1, 64),
      w4b, w4d, b4.reshape(1, 128))

    x1v = x1.reshape(B, P, 64)
    idx2 = pl.pallas_call(
        _knn2_body, grid=(B, nb),
        in_specs=[pl.BlockSpec((1, TPK, 64), lambda b, p: (b, p, 0)),
                  pl.BlockSpec((1, P, 64), lambda b, p: (b, 0, 0))],
        out_specs=pl.BlockSpec((TPK, KP), lambda b, p: (b * nb + p, 0)),
        out_shape=jax.ShapeDtypeStruct((NP, KP), jnp.int32),
    )(x1v, x1v)

    m = _sc_gather_max(y, idx2.reshape(-1))

    pooled = pl.pallas_call(
        _pool_body, grid=(B, P // TPP),
        in_specs=[pl.BlockSpec((TPP, 64), lambda b, p: (b * (P // TPP) + p, 0)),
                  pl.BlockSpec((TPP, 128), lambda b, p: (b * (P // TPP) + p, 0)),
                  pl.BlockSpec((TPP, 128), lambda b, p: (b * (P // TPP) + p, 0)),
                  pl.BlockSpec((64, 1024), lambda b, p: (0, 0)),
                  pl.BlockSpec((128, 1024), lambda b, p: (0, 0)),
                  pl.BlockSpec((1, 1024), lambda b, p: (0, 0))],
        out_specs=pl.BlockSpec((1, 1024), lambda b, p: (b, 0)),
        out_shape=jax.ShapeDtypeStruct((B, 1024), F32),
    )(x1, u, m, w5a, w5b, b5.reshape(1, 1024))

    out = pl.pallas_call(
        _head_body,
        in_specs=[pl.BlockSpec((B, 1024), lambda: (0, 0)),
                  pl.BlockSpec((1024, 512), lambda: (0, 0)),
                  pl.BlockSpec((1, 512), lambda: (0, 0)),
                  pl.BlockSpec((512, 256), lambda: (0, 0)),
                  pl.BlockSpec((1, 256), lambda: (0, 0)),
                  pl.BlockSpec((256, 40), lambda: (0, 0)),
                  pl.BlockSpec((1, 40), lambda: (0, 0))],
        out_specs=pl.BlockSpec((B, 40), lambda: (0, 0)),
        out_shape=jax.ShapeDtypeStruct((B, 40), F32),
    )(pooled, W6, b6.reshape(1, 512), W7, b7.reshape(1, 256),
      W8, b8.reshape(1, 40))
    return out


# speedup vs baseline: 7.1468x; 7.1468x over previous
import functools

import jax
import jax.numpy as jnp
from jax import lax
from jax.experimental import pallas as pl
from jax.experimental.pallas import tpu as pltpu
from jax.experimental.pallas import tpu_sc as plsc

B = 16
P = 1024
K = 20
KP = 24
NP = B * P
E = NP * K
EPS = 1e-5
F32 = jnp.float32

TPK = 256
TPE = 2048
NBP = NP // TPE
TPP = 256

_NC, _NS = 2, 16
_NW = _NC * _NS


def _popmin(d2, iota, nkeep):
    cols = []
    for _ in range(nkeep):
        m = jnp.min(d2, axis=1, keepdims=True)
        cand = jnp.where(d2 == m, iota, d2.shape[1])
        j = jnp.min(cand, axis=1, keepdims=True)
        cols.append(j)
        d2 = jnp.where(iota == j, jnp.inf, d2)
    return cols


BF = jnp.bfloat16


def _dot1x(a, b):
    return jnp.dot(a.astype(BF), b.astype(BF), preferred_element_type=F32)


def _knn1_body(posp_ref, post_ref, idx_ref):
    b = pl.program_id(0)
    x = posp_ref[...]
    xt = post_ref[0]
    sq_r = jnp.sum(x * x, axis=1, keepdims=True)
    sq_c = jnp.sum(xt * xt, axis=0, keepdims=True)
    d2 = sq_r + sq_c - 2.0 * _dot1x(x, xt)
    iota = lax.broadcasted_iota(jnp.int32, (TPK, P), 1)
    cols = _popmin(d2, iota, K)
    idx_ref[...] = jnp.concatenate(cols, axis=1) + b * P


def _knn2_body(x_ref, xt_ref, idx_ref):
    b = pl.program_id(0)
    p = pl.program_id(1)
    x = x_ref[0]
    xt = xt_ref[0]
    sq_r = jnp.sum(x * x, axis=1, keepdims=True)
    sq_c = jnp.sum(xt * xt, axis=0, keepdims=True)
    d2 = sq_r + sq_c - 2.0 * _dot1x(x, xt)
    iota = lax.broadcasted_iota(jnp.int32, (TPK, P), 1)
    cols = _popmin(d2, iota, K)
    knn = jnp.concatenate(cols, axis=1) + b * P
    self_i = (b * P + p * TPK
              + lax.broadcasted_iota(jnp.int32, (TPK, 1), 0))
    pad = jnp.concatenate([self_i] * (KP - K), axis=1)
    idx_ref[...] = jnp.concatenate([knn, pad], axis=1)


def _sc_gather_rows(table, idx):
    ew = E // _NW
    ch = 128
    nch = ew // ch
    mesh = plsc.VectorSubcoreMesh(core_axis_name="c", subcore_axis_name="s")

    @functools.partial(
        pl.kernel, mesh=mesh,
        out_type=jax.ShapeDtypeStruct((E, 128), F32),
        scratch_types=[pltpu.VMEM((ch,), jnp.int32),
                       pltpu.VMEM((ch, 128), F32),
                       pltpu.SemaphoreType.DMA],
    )
    def run(table_hbm, idx_hbm, out_hbm, idx_v, rows_v, sem):
        wid = lax.axis_index("s") * _NC + lax.axis_index("c")
        base = wid * ew

        def body(i, carry):
            off = base + i * ch
            pltpu.sync_copy(idx_hbm.at[pl.ds(off, ch)], idx_v)
            pltpu.async_copy(table_hbm.at[idx_v], rows_v, sem).wait()
            pltpu.sync_copy(rows_v, out_hbm.at[pl.ds(off, ch)])
            return carry

        lax.fori_loop(0, nch, body, 0)

    return run(table, idx)


def _sc_gather_max(y, idx):
    pw = NP // _NW
    mesh = plsc.VectorSubcoreMesh(core_axis_name="c", subcore_axis_name="s")

    @functools.partial(
        pl.kernel, mesh=mesh,
        out_type=jax.ShapeDtypeStruct((NP, 128), F32),
        scratch_types=[pltpu.VMEM((KP,), jnp.int32),
                       pltpu.VMEM((KP, 128), F32),
                       pltpu.VMEM((128,), F32),
                       pltpu.SemaphoreType.DMA],
    )
    def run(y_hbm, idx_hbm, out_hbm, idx_v, rows_v, row_v, sem):
        wid = lax.axis_index("s") * _NC + lax.axis_index("c")
        base = wid * pw

        def body(p, carry):
            pt = base + p
            pltpu.sync_copy(idx_hbm.at[pl.ds(pt * KP, KP)], idx_v)
            pltpu.async_copy(y_hbm.at[idx_v], rows_v, sem).wait()
            for c in range(8):
                v = rows_v[0, pl.ds(c * 16, 16)]
                for r in range(1, KP):
                    v = jnp.maximum(v, rows_v[r, pl.ds(c * 16, 16)])
                row_v[pl.ds(c * 16, 16)] = v
            pltpu.sync_copy(row_v, out_hbm.at[pt])
            return carry

        lax.fori_loop(0, pw, body, 0)

    return run(y, idx)


def _stats_update(st_ref, h, g):
    st = jnp.concatenate([jnp.sum(h, axis=0, keepdims=True),
                          jnp.sum(h * h, axis=0, keepdims=True)], axis=0)

    @pl.when(g == 0)
    def _():
        st_ref[...] = st

    @pl.when(g != 0)
    def _():
        st_ref[...] = st_ref[...] + st


def _norm_consts(st):
    mu = st[0:1] * (1.0 / E)
    var = st[1:2] * (1.0 / E) - mu * mu
    return mu, lax.rsqrt(var + EPS)


def _edge1_body(xi_ref, ge_ref, wa_ref, wb_ref, b1_ref, h1_ref, st_ref):
    g = pl.program_id(0)
    xi = xi_ref[...]
    wa = wa_ref[...].astype(BF)
    wb = wb_ref[...].astype(BF)
    hi = jnp.dot(xi.astype(BF), wa, preferred_element_type=F32) + b1_ref[...]
    ssum = jnp.zeros((1, 64), F32)
    ssq = jnp.zeros((1, 64), F32)
    for k in range(K):
        xj = ge_ref[k][:, 0:16]
        h1k = hi + jnp.dot((xj - xi).astype(BF), wb, preferred_element_type=F32)
        h1_ref[k] = h1k
        ssum = ssum + jnp.sum(h1k, axis=0, keepdims=True)
        ssq = ssq + jnp.sum(h1k * h1k, axis=0, keepdims=True)
    st = jnp.concatenate([ssum, ssq], axis=0)

    @pl.when(g == 0)
    def _():
        st_ref[...] = st

    @pl.when(g != 0)
    def _():
        st_ref[...] = st_ref[...] + st


def _edge2_body(h1_ref, st1_ref, w2_ref, b2_ref, h2_ref, st_ref):
    g = pl.program_id(0)
    mu, rs = _norm_consts(st1_ref[...])
    hn = jnp.maximum((h1_ref[...] - mu) * rs, 0.0)
    h2 = _dot1x(hn, w2_ref[...]) + b2_ref[...]
    h2_ref[...] = h2
    _stats_update(st_ref, h2, g)


def _edge3_body(h2_ref, st2_ref, w3_ref, b3_ref, w4b_ref, w4d_ref, b4_ref,
                x1_ref, y_ref, u_ref):
    mu, rs = _norm_consts(st2_ref[...])
    acc = jnp.full((TPP, 64), -jnp.inf, F32)
    for k in range(K):
        hn = jnp.maximum((h2_ref[k] - mu) * rs, 0.0)
        v = _dot1x(hn, w3_ref[...]) + b3_ref[...]
        acc = jnp.maximum(acc, v)
    x1_ref[...] = acc
    y_ref[...] = _dot1x(acc, w4b_ref[...])
    u_ref[...] = _dot1x(acc, w4d_ref[...]) + b4_ref[...]


def _pool_body(x1_ref, u_ref, m_ref, w5a_ref, w5b_ref, b5_ref, out_ref):
    p = pl.program_id(1)
    t = (_dot1x(x1_ref[...], w5a_ref[...])
         + _dot1x(u_ref[...] + m_ref[...], w5b_ref[...])
         + b5_ref[...])
    v = jnp.broadcast_to(jnp.max(t, axis=0, keepdims=True), (8, 1024))[None]

    @pl.when(p == 0)
    def _():
        out_ref[...] = v

    @pl.when(p != 0)
    def _():
        out_ref[...] = jnp.maximum(out_ref[...], v)


def _bn_relu_rows(h):
    mu = jnp.mean(h, axis=0, keepdims=True)
    var = jnp.mean((h - mu) ** 2, axis=0, keepdims=True)
    return jnp.maximum((h - mu) * lax.rsqrt(var + EPS), 0.0)


def _head_body(z_ref, w6_ref, b6_ref, w7_ref, b7_ref, w8_ref, b8_ref, o_ref):
    h = _dot1x(z_ref[...], w6_ref[...]) + b6_ref[...]
    h = _bn_relu_rows(h)
    h = _dot1x(h, w7_ref[...]) + b7_ref[...]
    h = _bn_relu_rows(h)
    h = _dot1x(h, w8_ref[...]) + b8_ref[...]
    mx = jnp.max(h, axis=1, keepdims=True)
    e = jnp.exp(h - mx)
    o_ref[...] = h - mx - jnp.log(jnp.sum(e, axis=1, keepdims=True))


def kernel(pos, batch, W1, b1, W2, b2, W3, b3, W4, b4, W5, b5, W6, b6, W7, b7, W8, b8):
    del batch
    posp = jnp.pad(pos, ((0, 0), (0, 13)))
    post = jnp.pad(pos.reshape(B, P, 3).transpose(0, 2, 1),
                   ((0, 0), (0, 13), (0, 0)))
    w1a = jnp.pad(W1[0:3], ((0, 13), (0, 0)))
    w1b = jnp.pad(W1[3:6], ((0, 13), (0, 0)))
    w4a, w4b = W4[:64], W4[64:]
    w4d = w4a - w4b
    w5a, w5b = W5[:64], W5[64:]

    nb = P // TPK

    idx1 = pl.pallas_call(
        _knn1_body, grid=(B, nb),
        in_specs=[pl.BlockSpec((TPK, 16), lambda b, p: (b * nb + p, 0)),
                  pl.BlockSpec((1, 16, P), lambda b, p: (b, 0, 0))],
        out_specs=pl.BlockSpec((TPK, K), lambda b, p: (b * nb + p, 0)),
        out_shape=jax.ShapeDtypeStruct((NP, K), jnp.int32),
    )(posp, post)

    idx_sm = idx1.T.reshape(-1)
    gtab = jnp.pad(pos, ((0, 0), (0, 125)))
    ge = _sc_gather_rows(gtab, idx_sm)

    h1, st1 = pl.pallas_call(
        _edge1_body, grid=(NP // TPP,),
        in_specs=[pl.BlockSpec((TPP, 16), lambda p: (p, 0)),
                  pl.BlockSpec((K, TPP, 128), lambda p: (0, p, 0)),
                  pl.BlockSpec((16, 64), lambda p: (0, 0)),
                  pl.BlockSpec((16, 64), lambda p: (0, 0)),
                  pl.BlockSpec((1, 64), lambda p: (0, 0))],
        out_specs=[pl.BlockSpec((K, TPP, 64), lambda p: (0, p, 0)),
                   pl.BlockSpec((2, 64), lambda p: (0, 0))],
        out_shape=[jax.ShapeDtypeStruct((K, NP, 64), F32),
                   jax.ShapeDtypeStruct((2, 64), F32)],
    )(posp, ge.reshape(K, NP, 128), w1a, w1b, b1.reshape(1, 64))
    h1 = h1.reshape(E, 64)

    h2, st2 = pl.pallas_call(
        _edge2_body, grid=(E // TPE,),
        in_specs=[pl.BlockSpec((TPE, 64), lambda g: (g, 0)),
                  pl.BlockSpec((2, 64), lambda g: (0, 0)),
                  pl.BlockSpec((64, 64), lambda g: (0, 0)),
                  pl.BlockSpec((1, 64), lambda g: (0, 0))],
        out_specs=[pl.BlockSpec((TPE, 64), lambda g: (g, 0)),
                   pl.BlockSpec((2, 64), lambda g: (0, 0))],
        out_shape=[jax.ShapeDtypeStruct((E, 64), F32),
                   jax.ShapeDtypeStruct((2, 64), F32)],
    )(h1, st1, W2, b2.reshape(1, 64))

    x1, y, u = pl.pallas_call(
        _edge3_body, grid=(NP // TPP,),
        in_specs=[pl.BlockSpec((K, TPP, 64), lambda p: (0, p, 0)),
                  pl.BlockSpec((2, 64), lambda p: (0, 0)),
                  pl.BlockSpec((64, 64), lambda p: (0, 0)),
                  pl.BlockSpec((1, 64), lambda p: (0, 0)),
                  pl.BlockSpec((64, 128), lambda p: (0, 0)),
                  pl.BlockSpec((64, 128), lambda p: (0, 0)),
                  pl.BlockSpec((1, 128), lambda p: (0, 0))],
        out_specs=[pl.BlockSpec((TPP, 64), lambda p: (p, 0)),
                   pl.BlockSpec((TPP, 128), lambda p: (p, 0)),
                   pl.BlockSpec((TPP, 128), lambda p: (p, 0))],
        out_shape=[jax.ShapeDtypeStruct((NP, 64), F32),
                   jax.ShapeDtypeStruct((NP, 128), F32),
                   jax.ShapeDtypeStruct((NP, 128), F32)],
    )(h2.reshape(K, NP, 64), st2, W3, b3.reshape(1, 64),
      w4b, w4d, b4.reshape(1, 128))

    x1v = x1.reshape(B, P, 64)
    x1t = x1v.transpose(0, 2, 1)
    idx2 = pl.pallas_call(
        _knn2_body, grid=(B, nb),
        in_specs=[pl.BlockSpec((1, TPK, 64), lambda b, p: (b, p, 0)),
                  pl.BlockSpec((1, 64, P), lambda b, p: (b, 0, 0))],
        out_specs=pl.BlockSpec((TPK, KP), lambda b, p: (b * nb + p, 0)),
        out_shape=jax.ShapeDtypeStruct((NP, KP), jnp.int32),
    )(x1v, x1t)

    m = _sc_gather_max(y, idx2.reshape(-1))

    pooled = pl.pallas_call(
        _pool_body, grid=(B, P // TPP),
        in_specs=[pl.BlockSpec((TPP, 64), lambda b, p: (b * (P // TPP) + p, 0)),
                  pl.BlockSpec((TPP, 128), lambda b, p: (b * (P // TPP) + p, 0)),
                  pl.BlockSpec((TPP, 128), lambda b, p: (b * (P // TPP) + p, 0)),
                  pl.BlockSpec((64, 1024), lambda b, p: (0, 0)),
                  pl.BlockSpec((128, 1024), lambda b, p: (0, 0)),
                  pl.BlockSpec((1, 1024), lambda b, p: (0, 0))],
        out_specs=pl.BlockSpec((1, 8, 1024), lambda b, p: (b, 0, 0)),
        out_shape=jax.ShapeDtypeStruct((B, 8, 1024), F32),
    )(x1, u, m, w5a, w5b, b5.reshape(1, 1024))
    pooled = pooled[:, 0, :]

    out = pl.pallas_call(
        _head_body,
        in_specs=[pl.BlockSpec((B, 1024), lambda: (0, 0)),
                  pl.BlockSpec((1024, 512), lambda: (0, 0)),
                  pl.BlockSpec((1, 512), lambda: (0, 0)),
                  pl.BlockSpec((512, 256), lambda: (0, 0)),
                  pl.BlockSpec((1, 256), lambda: (0, 0)),
                  pl.BlockSpec((256, 40), lambda: (0, 0)),
                  pl.BlockSpec((1, 40), lambda: (0, 0))],
        out_specs=pl.BlockSpec((B, 40), lambda: (0, 0)),
        out_shape=jax.ShapeDtypeStruct((B, 40), F32),
    )(pooled, W6, b6.reshape(1, 512), W7, b7.reshape(1, 256),
      W8, b8.reshape(1, 40))
    return out


# speedup vs baseline: 8.7403x; 1.2230x over previous
import functools

import jax
import jax.numpy as jnp
from jax import lax
from jax.experimental import pallas as pl
from jax.experimental.pallas import tpu as pltpu
from jax.experimental.pallas import tpu_sc as plsc

B = 16
P = 1024
K = 20
NP = B * P
E = NP * K
EPS = 1e-5
F32 = jnp.float32

TPK = 256
TPE = 2048
NBP = NP // TPE
TPP = 256

_NC, _NS = 2, 16
_NW = _NC * _NS


def _popmin(d2, iota, nkeep):
    cols = []
    for _ in range(nkeep):
        m = jnp.min(d2, axis=1, keepdims=True)
        cand = jnp.where(d2 == m, iota, d2.shape[1])
        j = jnp.min(cand, axis=1, keepdims=True)
        cols.append(j)
        d2 = jnp.where(iota == j, jnp.inf, d2)
    return cols


BF = jnp.bfloat16


def _dot1x(a, b):
    return jnp.dot(a.astype(BF), b.astype(BF), preferred_element_type=F32)


def _knn1_body(posp_ref, post_ref, posb_ref, xjp_ref):
    x = posp_ref[...]
    xt = post_ref[0]
    pb = posb_ref[...]
    hi = pb.astype(BF)
    r1 = pb - hi.astype(F32)
    mid = r1.astype(BF)
    lo = (r1 - mid.astype(F32)).astype(BF)
    sq_r = jnp.sum(x * x, axis=1, keepdims=True)
    sq_c = jnp.sum(xt * xt, axis=0, keepdims=True)
    d2 = sq_r + sq_c - 2.0 * _dot1x(x, xt)
    iota = lax.broadcasted_iota(jnp.int32, (TPK, P), 1)
    xjs = []
    for _ in range(K):
        m = jnp.min(d2, axis=1, keepdims=True)
        cand = jnp.where(d2 == m, iota, P)
        j = jnp.min(cand, axis=1, keepdims=True)
        sel = iota == j
        sb = sel.astype(BF)
        xj = (jnp.dot(sb, hi, preferred_element_type=F32)
              + jnp.dot(sb, mid, preferred_element_type=F32)
              + jnp.dot(sb, lo, preferred_element_type=F32))
        xjs.append(xj)
        d2 = jnp.where(sel, jnp.inf, d2)
    xjp_ref[...] = jnp.concatenate(xjs, axis=1)


def _knn2_body(x_ref, xt_ref, idx_ref):
    b = pl.program_id(0)
    p = pl.program_id(1)
    x = x_ref[0]
    xt = xt_ref[0]
    sq_r = jnp.sum(x * x, axis=1, keepdims=True)
    sq_c = jnp.sum(xt * xt, axis=0, keepdims=True)
    d2 = sq_r + sq_c - 2.0 * _dot1x(x, xt)
    iota = lax.broadcasted_iota(jnp.int32, (TPK, P), 1)
    cols = _popmin(d2, iota, K)
    idx_ref[...] = jnp.concatenate(cols, axis=1) + b * P


G4 = 4


def _sc_gather_max(y, idx):
    pw = NP // _NW
    ngrp = pw // G4
    gi = G4 * K
    mesh = plsc.VectorSubcoreMesh(core_axis_name="c", subcore_axis_name="s")

    @functools.partial(
        pl.kernel, mesh=mesh,
        out_type=jax.ShapeDtypeStruct((NP, 128), F32),
        scratch_types=[pltpu.VMEM((pw * K,), jnp.int32),
                       pltpu.VMEM((gi, 128), F32),
                       pltpu.VMEM((gi, 128), F32),
                       pltpu.VMEM((G4, 128), F32),
                       pltpu.SemaphoreType.DMA,
                       pltpu.SemaphoreType.DMA],
    )
    def run(y_hbm, idx_hbm, out_hbm, idx_all, rows0, rows1, out_v, sem0, sem1):
        wid = lax.axis_index("s") * _NC + lax.axis_index("c")
        base = wid * pw
        pltpu.sync_copy(idx_hbm.at[pl.ds(base * K, pw * K)], idx_all)
        pltpu.async_copy(y_hbm.at[idx_all.at[pl.ds(0, gi)]], rows0, sem0)
        pltpu.async_copy(y_hbm.at[idx_all.at[pl.ds(gi, gi)]], rows1, sem1)

        def half(g, rows_v, sem):
            pltpu.make_async_copy(y_hbm.at[idx_all.at[pl.ds(0, gi)]],
                                  rows_v, sem).wait()
            for i in range(G4):
                for c in range(8):
                    v = rows_v[i * K, pl.ds(c * 16, 16)]
                    for r in range(1, K):
                        v = jnp.maximum(v, rows_v[i * K + r, pl.ds(c * 16, 16)])
                    out_v[i, pl.ds(c * 16, 16)] = v
            pltpu.sync_copy(out_v, out_hbm.at[pl.ds(base + g * G4, G4)])
            nxt = g + 2

            @pl.when(nxt < ngrp)
            def _():
                pltpu.async_copy(y_hbm.at[idx_all.at[pl.ds(nxt * gi, gi)]],
                                 rows_v, sem)

        def body(gg, carry):
            half(2 * gg, rows0, sem0)
            half(2 * gg + 1, rows1, sem1)
            return carry

        lax.fori_loop(0, ngrp // 2, body, 0)

    return run(y, idx)


def _stats_update(st_ref, h, g):
    st = jnp.concatenate([jnp.sum(h, axis=0, keepdims=True),
                          jnp.sum(h * h, axis=0, keepdims=True)], axis=0)

    @pl.when(g == 0)
    def _():
        st_ref[...] = st

    @pl.when(g != 0)
    def _():
        st_ref[...] = st_ref[...] + st


def _norm_consts(st):
    mu = st[0:1] * (1.0 / E)
    var = st[1:2] * (1.0 / E) - mu * mu
    return mu, lax.rsqrt(var + EPS)


def _edge1_body(xi_ref, xjp_ref, wa_ref, wb_ref, b1_ref, h1_ref, st_ref):
    g = pl.program_id(0)
    xi = xi_ref[...]
    wa = wa_ref[...].astype(BF)
    wb = wb_ref[...].astype(BF)
    hi = jnp.dot(xi.astype(BF), wa, preferred_element_type=F32) + b1_ref[...]
    ssum = jnp.zeros((1, 64), F32)
    ssq = jnp.zeros((1, 64), F32)
    for k in range(K):
        xj = xjp_ref[:, k * 16:(k + 1) * 16]
        h1k = hi + jnp.dot((xj - xi).astype(BF), wb, preferred_element_type=F32)
        h1_ref[k] = h1k
        ssum = ssum + jnp.sum(h1k, axis=0, keepdims=True)
        ssq = ssq + jnp.sum(h1k * h1k, axis=0, keepdims=True)
    st = jnp.concatenate([ssum, ssq], axis=0)

    @pl.when(g == 0)
    def _():
        st_ref[...] = st

    @pl.when(g != 0)
    def _():
        st_ref[...] = st_ref[...] + st


def _edge2_body(h1_ref, st1_ref, w2_ref, b2_ref, h2_ref, st_ref):
    g = pl.program_id(0)
    mu, rs = _norm_consts(st1_ref[...])
    hn = jnp.maximum((h1_ref[...] - mu) * rs, 0.0)
    h2 = _dot1x(hn, w2_ref[...]) + b2_ref[...]
    h2_ref[...] = h2
    _stats_update(st_ref, h2, g)


def _edge3_body(h2_ref, st2_ref, w3_ref, b3_ref, w4b_ref, w4d_ref, b4_ref,
                x1_ref, y_ref, u_ref):
    mu, rs = _norm_consts(st2_ref[...])
    acc = jnp.full((TPP, 64), -jnp.inf, F32)
    for k in range(K):
        hn = jnp.maximum((h2_ref[k] - mu) * rs, 0.0)
        v = _dot1x(hn, w3_ref[...]) + b3_ref[...]
        acc = jnp.maximum(acc, v)
    x1_ref[...] = acc
    y_ref[...] = _dot1x(acc, w4b_ref[...])
    u_ref[...] = _dot1x(acc, w4d_ref[...]) + b4_ref[...]


def _pool_body(x1_ref, u_ref, m_ref, w5a_ref, w5b_ref, b5_ref, out_ref):
    p = pl.program_id(1)
    t = (_dot1x(x1_ref[...], w5a_ref[...])
         + _dot1x(u_ref[...] + m_ref[...], w5b_ref[...])
         + b5_ref[...])
    v = jnp.broadcast_to(jnp.max(t, axis=0, keepdims=True), (8, 1024))[None]

    @pl.when(p == 0)
    def _():
        out_ref[...] = v

    @pl.when(p != 0)
    def _():
        out_ref[...] = jnp.maximum(out_ref[...], v)


def _bn_relu_rows(h):
    mu = jnp.mean(h, axis=0, keepdims=True)
    var = jnp.mean((h - mu) ** 2, axis=0, keepdims=True)
    return jnp.maximum((h - mu) * lax.rsqrt(var + EPS), 0.0)


def _head_body(z_ref, w6_ref, b6_ref, w7_ref, b7_ref, w8_ref, b8_ref, o_ref):
    h = _dot1x(z_ref[...], w6_ref[...]) + b6_ref[...]
    h = _bn_relu_rows(h)
    h = _dot1x(h, w7_ref[...]) + b7_ref[...]
    h = _bn_relu_rows(h)
    h = _dot1x(h, w8_ref[...]) + b8_ref[...]
    mx = jnp.max(h, axis=1, keepdims=True)
    e = jnp.exp(h - mx)
    o_ref[...] = h - mx - jnp.log(jnp.sum(e, axis=1, keepdims=True))


def kernel(pos, batch, W1, b1, W2, b2, W3, b3, W4, b4, W5, b5, W6, b6, W7, b7, W8, b8):
    del batch
    posp = jnp.pad(pos, ((0, 0), (0, 13)))
    post = jnp.pad(pos.reshape(B, P, 3).transpose(0, 2, 1),
                   ((0, 0), (0, 13), (0, 0)))
    w1a = jnp.pad(W1[0:3], ((0, 13), (0, 0)))
    w1b = jnp.pad(W1[3:6], ((0, 13), (0, 0)))
    w4a, w4b = W4[:64], W4[64:]
    w4d = w4a - w4b
    w5a, w5b = W5[:64], W5[64:]

    nb = P // TPK

    xjp = pl.pallas_call(
        _knn1_body, grid=(B, nb),
        in_specs=[pl.BlockSpec((TPK, 16), lambda b, p: (b * nb + p, 0)),
                  pl.BlockSpec((1, 16, P), lambda b, p: (b, 0, 0)),
                  pl.BlockSpec((P, 16), lambda b, p: (b, 0))],
        out_specs=pl.BlockSpec((TPK, K * 16), lambda b, p: (b * nb + p, 0)),
        out_shape=jax.ShapeDtypeStruct((NP, K * 16), F32),
    )(posp, post, posp)

    h1, st1 = pl.pallas_call(
        _edge1_body, grid=(NP // TPP,),
        in_specs=[pl.BlockSpec((TPP, 16), lambda p: (p, 0)),
                  pl.BlockSpec((TPP, K * 16), lambda p: (p, 0)),
                  pl.BlockSpec((16, 64), lambda p: (0, 0)),
                  pl.BlockSpec((16, 64), lambda p: (0, 0)),
                  pl.BlockSpec((1, 64), lambda p: (0, 0))],
        out_specs=[pl.BlockSpec((K, TPP, 64), lambda p: (0, p, 0)),
                   pl.BlockSpec((2, 64), lambda p: (0, 0))],
        out_shape=[jax.ShapeDtypeStruct((K, NP, 64), F32),
                   jax.ShapeDtypeStruct((2, 64), F32)],
    )(posp, xjp, w1a, w1b, b1.reshape(1, 64))
    h1 = h1.reshape(E, 64)

    h2, st2 = pl.pallas_call(
        _edge2_body, grid=(E // TPE,),
        in_specs=[pl.BlockSpec((TPE, 64), lambda g: (g, 0)),
                  pl.BlockSpec((2, 64), lambda g: (0, 0)),
                  pl.BlockSpec((64, 64), lambda g: (0, 0)),
                  pl.BlockSpec((1, 64), lambda g: (0, 0))],
        out_specs=[pl.BlockSpec((TPE, 64), lambda g: (g, 0)),
                   pl.BlockSpec((2, 64), lambda g: (0, 0))],
        out_shape=[jax.ShapeDtypeStruct((E, 64), F32),
                   jax.ShapeDtypeStruct((2, 64), F32)],
    )(h1, st1, W2, b2.reshape(1, 64))

    x1, y, u = pl.pallas_call(
        _edge3_body, grid=(NP // TPP,),
        in_specs=[pl.BlockSpec((K, TPP, 64), lambda p: (0, p, 0)),
                  pl.BlockSpec((2, 64), lambda p: (0, 0)),
                  pl.BlockSpec((64, 64), lambda p: (0, 0)),
                  pl.BlockSpec((1, 64), lambda p: (0, 0)),
                  pl.BlockSpec((64, 128), lambda p: (0, 0)),
                  pl.BlockSpec((64, 128), lambda p: (0, 0)),
                  pl.BlockSpec((1, 128), lambda p: (0, 0))],
        out_specs=[pl.BlockSpec((TPP, 64), lambda p: (p, 0)),
                   pl.BlockSpec((TPP, 128), lambda p: (p, 0)),
                   pl.BlockSpec((TPP, 128), lambda p: (p, 0))],
        out_shape=[jax.ShapeDtypeStruct((NP, 64), F32),
                   jax.ShapeDtypeStruct((NP, 128), F32),
                   jax.ShapeDtypeStruct((NP, 128), F32)],
    )(h2.reshape(K, NP, 64), st2, W3, b3.reshape(1, 64),
      w4b, w4d, b4.reshape(1, 128))

    x1v = x1.reshape(B, P, 64)
    x1t = x1v.transpose(0, 2, 1)
    idx2 = pl.pallas_call(
        _knn2_body, grid=(B, nb),
        in_specs=[pl.BlockSpec((1, TPK, 64), lambda b, p: (b, p, 0)),
                  pl.BlockSpec((1, 64, P), lambda b, p: (b, 0, 0))],
        out_specs=pl.BlockSpec((TPK, K), lambda b, p: (b * nb + p, 0)),
        out_shape=jax.ShapeDtypeStruct((NP, K), jnp.int32),
    )(x1v, x1t)

    m = _sc_gather_max(y, idx2.reshape(-1))

    pooled = pl.pallas_call(
        _pool_body, grid=(B, P // TPP),
        in_specs=[pl.BlockSpec((TPP, 64), lambda b, p: (b * (P // TPP) + p, 0)),
                  pl.BlockSpec((TPP, 128), lambda b, p: (b * (P // TPP) + p, 0)),
                  pl.BlockSpec((TPP, 128), lambda b, p: (b * (P // TPP) + p, 0)),
                  pl.BlockSpec((64, 1024), lambda b, p: (0, 0)),
                  pl.BlockSpec((128, 1024), lambda b, p: (0, 0)),
                  pl.BlockSpec((1, 1024), lambda b, p: (0, 0))],
        out_specs=pl.BlockSpec((1, 8, 1024), lambda b, p: (b, 0, 0)),
        out_shape=jax.ShapeDtypeStruct((B, 8, 1024), F32),
    )(x1, u, m, w5a, w5b, b5.reshape(1, 1024))
    pooled = pooled[:, 0, :]

    out = pl.pallas_call(
        _head_body,
        in_specs=[pl.BlockSpec((B, 1024), lambda: (0, 0)),
                  pl.BlockSpec((1024, 512), lambda: (0, 0)),
                  pl.BlockSpec((1, 512), lambda: (0, 0)),
                  pl.BlockSpec((512, 256), lambda: (0, 0)),
                  pl.BlockSpec((1, 256), lambda: (0, 0)),
                  pl.BlockSpec((256, 40), lambda: (0, 0)),
                  pl.BlockSpec((1, 40), lambda: (0, 0))],
        out_specs=pl.BlockSpec((B, 40), lambda: (0, 0)),
        out_shape=jax.ShapeDtypeStruct((B, 40), F32),
    )(pooled, W6, b6.reshape(1, 512), W7, b7.reshape(1, 256),
      W8, b8.reshape(1, 40))
    return out


# speedup vs baseline: 12.5954x; 1.4411x over previous
import functools

import jax
import jax.numpy as jnp
from jax import lax
from jax.experimental import pallas as pl
from jax.experimental.pallas import tpu as pltpu
from jax.experimental.pallas import tpu_sc as plsc

B = 16
P = 1024
K = 20
NP = B * P
E = NP * K
EPS = 1e-5
F32 = jnp.float32

TPK = 512
TPE = 4096
NBP = NP // TPE
TPP = 512

_NC, _NS = 2, 16
_NW = _NC * _NS


def _popmin(d2, iota, nkeep):
    n = float(d2.shape[1])
    cols = []
    for _ in range(nkeep):
        m = jnp.min(d2, axis=1, keepdims=True)
        cand = jnp.where(d2 == m, iota, n)
        j = jnp.min(cand, axis=1, keepdims=True)
        cols.append(j.astype(jnp.int32))
        d2 = jnp.where(iota == j, jnp.inf, d2)
    return cols


BF = jnp.bfloat16


def _dot1x(a, b):
    return jnp.dot(a.astype(BF), b.astype(BF), preferred_element_type=F32)


def _knn1_body(posp_ref, post_ref, posb_ref, xjp_ref):
    x = posp_ref[...]
    xt = post_ref[0]
    pb = posb_ref[...]
    hi = pb.astype(BF)
    r1 = pb - hi.astype(F32)
    mid = r1.astype(BF)
    lo = (r1 - mid.astype(F32)).astype(BF)
    tab = jnp.concatenate([hi, mid, lo], axis=1)
    sq_r = jnp.sum(x * x, axis=1, keepdims=True)
    sq_c = jnp.sum(xt * xt, axis=0, keepdims=True)
    d2 = sq_r + sq_c - 2.0 * _dot1x(x, xt)
    iota = lax.broadcasted_iota(jnp.int32, (TPK, P), 1).astype(F32)
    xjs = []
    for _ in range(K):
        m = jnp.min(d2, axis=1, keepdims=True)
        cand = jnp.where(d2 == m, iota, float(P))
        j = jnp.min(cand, axis=1, keepdims=True)
        sel = iota == j
        xq = jnp.dot(sel.astype(BF), tab, preferred_element_type=F32)
        xjs.append(xq[:, 0:16] + xq[:, 16:32] + xq[:, 32:48])
        d2 = jnp.where(sel, jnp.inf, d2)
    xjp_ref[...] = jnp.concatenate(xjs, axis=1)


def _knn2_body(x_ref, xt_ref, idx_ref):
    b = pl.program_id(0)
    p = pl.program_id(1)
    x = x_ref[0]
    xt = xt_ref[0]
    sq_r = jnp.sum(x * x, axis=1, keepdims=True)
    sq_c = jnp.sum(xt * xt, axis=0, keepdims=True)
    d2 = sq_r + sq_c - 2.0 * _dot1x(x, xt)
    iota = lax.broadcasted_iota(jnp.int32, (TPK, P), 1).astype(F32)
    cols = _popmin(d2, iota, K)
    idx_ref[...] = jnp.concatenate(cols, axis=1) + b * P


G4 = 4


def _sc_gather_max(y, idx):
    pw = NP // _NW
    ngrp = pw // G4
    gi = G4 * K
    mesh = plsc.VectorSubcoreMesh(core_axis_name="c", subcore_axis_name="s")

    @functools.partial(
        pl.kernel, mesh=mesh,
        out_type=jax.ShapeDtypeStruct((NP, 128), F32),
        scratch_types=[pltpu.VMEM((pw * K,), jnp.int32),
                       pltpu.VMEM((gi, 128), F32),
                       pltpu.VMEM((gi, 128), F32),
                       pltpu.VMEM((G4, 128), F32),
                       pltpu.SemaphoreType.DMA,
                       pltpu.SemaphoreType.DMA],
    )
    def run(y_hbm, idx_hbm, out_hbm, idx_all, rows0, rows1, out_v, sem0, sem1):
        wid = lax.axis_index("s") * _NC + lax.axis_index("c")
        base = wid * pw
        pltpu.sync_copy(idx_hbm.at[pl.ds(base * K, pw * K)], idx_all)
        pltpu.async_copy(y_hbm.at[idx_all.at[pl.ds(0, gi)]], rows0, sem0)
        pltpu.async_copy(y_hbm.at[idx_all.at[pl.ds(gi, gi)]], rows1, sem1)

        def half(g, rows_v, sem):
            pltpu.make_async_copy(y_hbm.at[idx_all.at[pl.ds(0, gi)]],
                                  rows_v, sem).wait()
            for i in range(G4):
                for c in range(8):
                    v = rows_v[i * K, pl.ds(c * 16, 16)]
                    for r in range(1, K):
                        v = jnp.maximum(v, rows_v[i * K + r, pl.ds(c * 16, 16)])
                    out_v[i, pl.ds(c * 16, 16)] = v
            pltpu.sync_copy(out_v, out_hbm.at[pl.ds(base + g * G4, G4)])
            nxt = g + 2

            @pl.when(nxt < ngrp)
            def _():
                pltpu.async_copy(y_hbm.at[idx_all.at[pl.ds(nxt * gi, gi)]],
                                 rows_v, sem)

        def body(gg, carry):
            half(2 * gg, rows0, sem0)
            half(2 * gg + 1, rows1, sem1)
            return carry

        lax.fori_loop(0, ngrp // 2, body, 0)

    return run(y, idx)


def _stats_update(st_ref, h, g):
    st = jnp.concatenate([jnp.sum(h, axis=0, keepdims=True),
                          jnp.sum(h * h, axis=0, keepdims=True)], axis=0)

    @pl.when(g == 0)
    def _():
        st_ref[...] = st

    @pl.when(g != 0)
    def _():
        st_ref[...] = st_ref[...] + st


def _norm_consts(st):
    mu = st[0:1] * (1.0 / E)
    var = st[1:2] * (1.0 / E) - mu * mu
    return mu, lax.rsqrt(var + EPS)


def _edge1_body(xi_ref, xjp_ref, wa_ref, wb_ref, b1_ref, h1_ref, st_ref):
    g = pl.program_id(0)
    xi = xi_ref[...]
    wa = wa_ref[...].astype(BF)
    wb = wb_ref[...].astype(BF)
    hi = jnp.dot(xi.astype(BF), wa, preferred_element_type=F32) + b1_ref[...]
    ssum = jnp.zeros((1, 64), F32)
    ssq = jnp.zeros((1, 64), F32)
    for k in range(K):
        xj = xjp_ref[:, k * 16:(k + 1) * 16]
        h1k = hi + jnp.dot((xj - xi).astype(BF), wb, preferred_element_type=F32)
        h1_ref[k] = h1k
        ssum = ssum + jnp.sum(h1k, axis=0, keepdims=True)
        ssq = ssq + jnp.sum(h1k * h1k, axis=0, keepdims=True)
    st = jnp.concatenate([ssum, ssq], axis=0)

    @pl.when(g == 0)
    def _():
        st_ref[...] = st

    @pl.when(g != 0)
    def _():
        st_ref[...] = st_ref[...] + st


def _edge2_body(h1_ref, st1_ref, w2_ref, b2_ref, h2_ref, st_ref):
    g = pl.program_id(0)
    mu, rs = _norm_consts(st1_ref[...])
    hn = jnp.maximum((h1_ref[...] - mu) * rs, 0.0)
    h2 = _dot1x(hn, w2_ref[...]) + b2_ref[...]
    h2_ref[...] = h2
    _stats_update(st_ref, h2, g)


def _edge3_body(h2_ref, st2_ref, w3_ref, b3_ref, w4b_ref, w4d_ref, b4_ref,
                x1_ref, y_ref, u_ref):
    mu, rs = _norm_consts(st2_ref[...])
    acc = jnp.full((TPP, 64), -jnp.inf, F32)
    for k in range(K):
        hn = jnp.maximum((h2_ref[k] - mu) * rs, 0.0)
        v = _dot1x(hn, w3_ref[...]) + b3_ref[...]
        acc = jnp.maximum(acc, v)
    x1_ref[...] = acc
    y_ref[...] = _dot1x(acc, w4b_ref[...])
    u_ref[...] = _dot1x(acc, w4d_ref[...]) + b4_ref[...]


def _pool_body(x1_ref, u_ref, m_ref, w5a_ref, w5b_ref, b5_ref, out_ref):
    p = pl.program_id(1)
    t = (_dot1x(x1_ref[...], w5a_ref[...])
         + _dot1x(u_ref[...] + m_ref[...], w5b_ref[...])
         + b5_ref[...])
    v = jnp.broadcast_to(jnp.max(t, axis=0, keepdims=True), (8, 1024))[None]

    @pl.when(p == 0)
    def _():
        out_ref[...] = v

    @pl.when(p != 0)
    def _():
        out_ref[...] = jnp.maximum(out_ref[...], v)


def _bn_relu_rows(h):
    mu = jnp.mean(h, axis=0, keepdims=True)
    var = jnp.mean((h - mu) ** 2, axis=0, keepdims=True)
    return jnp.maximum((h - mu) * lax.rsqrt(var + EPS), 0.0)


def _head_body(z_ref, w6_ref, b6_ref, w7_ref, b7_ref, w8_ref, b8_ref, o_ref):
    h = _dot1x(z_ref[...], w6_ref[...]) + b6_ref[...]
    h = _bn_relu_rows(h)
    h = _dot1x(h, w7_ref[...]) + b7_ref[...]
    h = _bn_relu_rows(h)
    h = _dot1x(h, w8_ref[...]) + b8_ref[...]
    mx = jnp.max(h, axis=1, keepdims=True)
    e = jnp.exp(h - mx)
    o_ref[...] = h - mx - jnp.log(jnp.sum(e, axis=1, keepdims=True))


def kernel(pos, batch, W1, b1, W2, b2, W3, b3, W4, b4, W5, b5, W6, b6, W7, b7, W8, b8):
    del batch
    posp = jnp.pad(pos, ((0, 0), (0, 13)))
    post = jnp.pad(pos.reshape(B, P, 3).transpose(0, 2, 1),
                   ((0, 0), (0, 13), (0, 0)))
    w1a = jnp.pad(W1[0:3], ((0, 13), (0, 0)))
    w1b = jnp.pad(W1[3:6], ((0, 13), (0, 0)))
    w4a, w4b = W4[:64], W4[64:]
    w4d = w4a - w4b
    w5a, w5b = W5[:64], W5[64:]

    nb = P // TPK

    xjp = pl.pallas_call(
        _knn1_body, grid=(B, nb),
        in_specs=[pl.BlockSpec((TPK, 16), lambda b, p: (b * nb + p, 0)),
                  pl.BlockSpec((1, 16, P), lambda b, p: (b, 0, 0)),
                  pl.BlockSpec((P, 16), lambda b, p: (b, 0))],
        out_specs=pl.BlockSpec((TPK, K * 16), lambda b, p: (b * nb + p, 0)),
        out_shape=jax.ShapeDtypeStruct((NP, K * 16), F32),
    )(posp, post, posp)

    h1, st1 = pl.pallas_call(
        _edge1_body, grid=(NP // TPP,),
        in_specs=[pl.BlockSpec((TPP, 16), lambda p: (p, 0)),
                  pl.BlockSpec((TPP, K * 16), lambda p: (p, 0)),
                  pl.BlockSpec((16, 64), lambda p: (0, 0)),
                  pl.BlockSpec((16, 64), lambda p: (0, 0)),
                  pl.BlockSpec((1, 64), lambda p: (0, 0))],
        out_specs=[pl.BlockSpec((K, TPP, 64), lambda p: (0, p, 0)),
                   pl.BlockSpec((2, 64), lambda p: (0, 0))],
        out_shape=[jax.ShapeDtypeStruct((K, NP, 64), F32),
                   jax.ShapeDtypeStruct((2, 64), F32)],
    )(posp, xjp, w1a, w1b, b1.reshape(1, 64))
    h1 = h1.reshape(E, 64)

    h2, st2 = pl.pallas_call(
        _edge2_body, grid=(E // TPE,),
        in_specs=[pl.BlockSpec((TPE, 64), lambda g: (g, 0)),
                  pl.BlockSpec((2, 64), lambda g: (0, 0)),
                  pl.BlockSpec((64, 64), lambda g: (0, 0)),
                  pl.BlockSpec((1, 64), lambda g: (0, 0))],
        out_specs=[pl.BlockSpec((TPE, 64), lambda g: (g, 0)),
                   pl.BlockSpec((2, 64), lambda g: (0, 0))],
        out_shape=[jax.ShapeDtypeStruct((E, 64), F32),
                   jax.ShapeDtypeStruct((2, 64), F32)],
    )(h1, st1, W2, b2.reshape(1, 64))

    x1, y, u = pl.pallas_call(
        _edge3_body, grid=(NP // TPP,),
        in_specs=[pl.BlockSpec((K, TPP, 64), lambda p: (0, p, 0)),
                  pl.BlockSpec((2, 64), lambda p: (0, 0)),
                  pl.BlockSpec((64, 64), lambda p: (0, 0)),
                  pl.BlockSpec((1, 64), lambda p: (0, 0)),
                  pl.BlockSpec((64, 128), lambda p: (0, 0)),
                  pl.BlockSpec((64, 128), lambda p: (0, 0)),
                  pl.BlockSpec((1, 128), lambda p: (0, 0))],
        out_specs=[pl.BlockSpec((TPP, 64), lambda p: (p, 0)),
                   pl.BlockSpec((TPP, 128), lambda p: (p, 0)),
                   pl.BlockSpec((TPP, 128), lambda p: (p, 0))],
        out_shape=[jax.ShapeDtypeStruct((NP, 64), F32),
                   jax.ShapeDtypeStruct((NP, 128), F32),
                   jax.ShapeDtypeStruct((NP, 128), F32)],
    )(h2.reshape(K, NP, 64), st2, W3, b3.reshape(1, 64),
      w4b, w4d, b4.reshape(1, 128))

    x1v = x1.reshape(B, P, 64)
    x1t = x1v.transpose(0, 2, 1)
    idx2 = pl.pallas_call(
        _knn2_body, grid=(B, nb),
        in_specs=[pl.BlockSpec((1, TPK, 64), lambda b, p: (b, p, 0)),
                  pl.BlockSpec((1, 64, P), lambda b, p: (b, 0, 0))],
        out_specs=pl.BlockSpec((TPK, K), lambda b, p: (b * nb + p, 0)),
        out_shape=jax.ShapeDtypeStruct((NP, K), jnp.int32),
    )(x1v, x1t)

    m = _sc_gather_max(y, idx2.reshape(-1))

    pooled = pl.pallas_call(
        _pool_body, grid=(B, P // TPP),
        in_specs=[pl.BlockSpec((TPP, 64), lambda b, p: (b * (P // TPP) + p, 0)),
                  pl.BlockSpec((TPP, 128), lambda b, p: (b * (P // TPP) + p, 0)),
                  pl.BlockSpec((TPP, 128), lambda b, p: (b * (P // TPP) + p, 0)),
                  pl.BlockSpec((64, 1024), lambda b, p: (0, 0)),
                  pl.BlockSpec((128, 1024), lambda b, p: (0, 0)),
                  pl.BlockSpec((1, 1024), lambda b, p: (0, 0))],
        out_specs=pl.BlockSpec((1, 8, 1024), lambda b, p: (b, 0, 0)),
        out_shape=jax.ShapeDtypeStruct((B, 8, 1024), F32),
    )(x1, u, m, w5a, w5b, b5.reshape(1, 1024))
    pooled = pooled[:, 0, :]

    out = pl.pallas_call(
        _head_body,
        in_specs=[pl.BlockSpec((B, 1024), lambda: (0, 0)),
                  pl.BlockSpec((1024, 512), lambda: (0, 0)),
                  pl.BlockSpec((1, 512), lambda: (0, 0)),
                  pl.BlockSpec((512, 256), lambda: (0, 0)),
                  pl.BlockSpec((1, 256), lambda: (0, 0)),
                  pl.BlockSpec((256, 40), lambda: (0, 0)),
                  pl.BlockSpec((1, 40), lambda: (0, 0))],
        out_specs=pl.BlockSpec((B, 40), lambda: (0, 0)),
        out_shape=jax.ShapeDtypeStruct((B, 40), F32),
    )(pooled, W6, b6.reshape(1, 512), W7, b7.reshape(1, 256),
      W8, b8.reshape(1, 40))
    return out
